# baseline (speedup 1.0000x reference)
import functools

import jax
import jax.numpy as jnp
from jax.experimental import pallas as pl
from jax.experimental.pallas import tpu as pltpu

_S = 56
_BATCH = 256
_D = 30
_CELLS = _BATCH * _S * _S
_CORES = 2
_BLOCK_ROWS = 4096
_STEPS = _CELLS // (_CORES * _BLOCK_ROWS)


def _loss_kernel(p_ref, t_ref, o_ref, *, steps):
    j = pl.program_id(1)

    p = p_ref[...]
    t = t_ref[...]

    lane = jax.lax.broadcasted_iota(jnp.int32, (1, _D), 1)
    is_wh = (lane >= 2) & (lane <= 3)
    extra = jnp.where(lane < 4, 5.0, jnp.where(lane == 4, 0.5, 1.0)).astype(
        jnp.float32)
    base = jnp.where(lane == 4, 0.5, 0.0).astype(jnp.float32)

    pe = jnp.where(is_wh, jnp.sqrt(p), p)
    te = jnp.where(is_wh, jnp.sqrt(t), t)
    d = pe - te
    d2 = d * d

    obj = (t[:, 4:5] == 1.0).astype(jnp.float32)
    w = base + obj * extra
    partial = jnp.sum(w * d2, axis=0, keepdims=True)

    @pl.when(j == 0)
    def _init():
        o_ref[...] = jnp.zeros_like(o_ref)

    o_ref[0] += partial


def kernel(predictions, target):
    p2 = predictions.reshape(_CELLS, _D)
    t2 = target.reshape(_CELLS, _D)

    in_spec = pl.BlockSpec(
        (_BLOCK_ROWS, _D), lambda i, j: (i * _STEPS + j, 0))
    out_spec = pl.BlockSpec((1, 1, _D), lambda i, j: (i, 0, 0))

    partials = pl.pallas_call(
        functools.partial(_loss_kernel, steps=_STEPS),
        grid=(_CORES, _STEPS),
        in_specs=[in_spec, in_spec],
        out_specs=out_spec,
        out_shape=jax.ShapeDtypeStruct((_CORES, 1, _D), jnp.float32),
        compiler_params=pltpu.CompilerParams(
            dimension_semantics=("parallel", "arbitrary")),
        name="yolo_loss",
    )(p2, t2)

    return jnp.sum(partials) / _BATCH
